# baseline (speedup 1.0000x reference)
import functools

import jax
import jax.numpy as jnp
from jax import lax
from jax.experimental import pallas as pl
from jax.experimental.pallas import tpu as pltpu
from jax.experimental.pallas import tpu_sc as plsc

B, T, D, K = 4, 1024, 256, 8192
N = B * T
NH = N // 2
TB = 256
NSTEPS = NH // TB

NC, NS, L = 2, 16, 16
NW = NC * NS
BPW = NH // NW


def _c2_body(cb_ref, c2_out):
    c = cb_ref[...]
    c2_out[...] = jnp.sum(c * c, axis=1)[None, :]


def _make_vq_body(final):
    def _vq_body(prec_ref, z_ref, cb_ref, c2_ref, probs_in, plogp_in,
                 idx_out, aux1_out, aux2_out,
                 probs_acc, plogp_acc):
        i = pl.program_id(0)

        @pl.when(i == 0)
        def _init():
            probs_acc[...] = probs_in[...]
            plogp_acc[...] = plogp_in[...]

        prec = prec_ref[0, 0]
        z = z_ref[...]
        c = cb_ref[...]
        ones_d = jnp.ones((D, 1), jnp.float32)
        ones_k = jnp.ones((K, 1), jnp.float32)

        zc = lax.dot_general(z, c, (((1,), (1,)), ((), ())),
                             preferred_element_type=jnp.float32)
        z2 = lax.dot_general(z * z, ones_d, (((1,), (0,)), ((), ())),
                             preferred_element_type=jnp.float32)
        dist = z2 - 2.0 * zc + c2_ref[...]
        logits = -prec * dist

        m = jnp.max(logits, axis=1, keepdims=True)
        iota1 = lax.broadcasted_iota(jnp.int32, (1, K), 1)
        idx = jnp.min(jnp.where(logits == m, iota1, K), axis=1)
        idx_out[...] = idx.reshape(1, 1, TB)

        t = logits - m
        e = jnp.exp(t)
        et = e * t
        s = lax.dot_general(e, ones_k, (((1,), (0,)), ((), ())),
                            preferred_element_type=jnp.float32)
        set_ = lax.dot_general(et, ones_k, (((1,), (0,)), ((), ())),
                               preferred_element_type=jnp.float32)
        rinv = 1.0 / s
        row_kld = set_ * rinv - jnp.log(s) + jnp.log(float(K))

        plogp_acc[...] = plogp_acc[...] + jnp.sum(row_kld)
        probs_acc[...] += lax.dot_general(rinv, e, (((0,), (0,)), ((), ())),
                                          preferred_element_type=jnp.float32)

        @pl.when(i == NSTEPS - 1)
        def _fin():
            if final:
                avg = probs_acc[...] / float(N)
                aux1_out[...] = plogp_acc[...] / float(N)
                aux2_out[...] = jnp.zeros_like(aux2_out) + jnp.exp(
                    -jnp.sum(avg * jnp.log(avg + 1e-7)))
            else:
                aux1_out[...] = probs_acc[...]
                aux2_out[...] = plogp_acc[...]

    return _vq_body


def _tc_half(final):
    aux1_shape = (1, 1) if final else (1, K)
    body = _make_vq_body(final)

    def call(z_half, codebook, c2, prec, probs_in, plogp_in):
        return pl.pallas_call(
            body,
            grid=(NSTEPS,),
            in_specs=[
                pl.BlockSpec(memory_space=pltpu.SMEM),
                pl.BlockSpec((TB, D), lambda i: (i, 0)),
                pl.BlockSpec((K, D), lambda i: (0, 0)),
                pl.BlockSpec((1, K), lambda i: (0, 0)),
                pl.BlockSpec((1, K), lambda i: (0, 0)),
                pl.BlockSpec((1, 1), lambda i: (0, 0)),
            ],
            out_specs=[
                pl.BlockSpec((1, 1, TB), lambda i: (i, 0, 0)),
                pl.BlockSpec(aux1_shape, lambda i: (0, 0)),
                pl.BlockSpec((1, 1), lambda i: (0, 0)),
            ],
            out_shape=[
                jax.ShapeDtypeStruct((NSTEPS, 1, TB), jnp.int32),
                jax.ShapeDtypeStruct(aux1_shape, jnp.float32),
                jax.ShapeDtypeStruct((1, 1), jnp.float32),
            ],
            scratch_shapes=[
                pltpu.VMEM((1, K), jnp.float32),
                pltpu.VMEM((1, 1), jnp.float32),
            ],
            compiler_params=pltpu.CompilerParams(
                dimension_semantics=("arbitrary",),
            ),
        )(prec, z_half, codebook, c2, probs_in, plogp_in)

    return call


_tc_half_a = _tc_half(final=False)
_tc_half_b = _tc_half(final=True)


def _make_sc_gather():
    mesh = plsc.VectorSubcoreMesh(core_axis_name="c", subcore_axis_name="s")

    @functools.partial(
        pl.kernel, mesh=mesh,
        out_type=[
            jax.ShapeDtypeStruct((NH, D), jnp.float32),
            jax.ShapeDtypeStruct((NH, D), jnp.float32),
            jax.ShapeDtypeStruct((NH, D), jnp.float32),
        ],
        scratch_types=[
            pltpu.VMEM((BPW,), jnp.int32),
            pltpu.VMEM((BPW, D), jnp.float32),
            pltpu.VMEM((BPW, D), jnp.float32),
            pltpu.VMEM((BPW, D), jnp.float32),
            pltpu.SemaphoreType.DMA,
        ],
    )
    def k(cb_hbm, idx_hbm, zcur_hbm, zres_hbm,
          zq_out, zcur_out, zres_out,
          idx_v, rows_v, zcur_v, zres_v, sem):
        wid = lax.axis_index("s") * NC + lax.axis_index("c")
        base = wid * BPW
        pltpu.sync_copy(idx_hbm.at[pl.ds(base, BPW)], idx_v)
        cp = pltpu.async_copy(cb_hbm.at[idx_v], rows_v, sem)
        pltpu.sync_copy(zcur_hbm.at[pl.ds(base, BPW)], zcur_v)
        pltpu.sync_copy(zres_hbm.at[pl.ds(base, BPW)], zres_v)
        cp.wait()
        pltpu.sync_copy(rows_v, zq_out.at[pl.ds(base, BPW)])

        def row_body(r, carry):
            for cc in range(D // L):
                sl = pl.ds(cc * L, L)
                zq = rows_v[r, sl]
                zcur_v[r, sl] = zcur_v[r, sl] + zq
                zres_v[r, sl] = zres_v[r, sl] - zq
            return carry

        lax.fori_loop(0, BPW, row_body, 0)
        pltpu.sync_copy(zcur_v, zcur_out.at[pl.ds(base, BPW)])
        pltpu.sync_copy(zres_v, zres_out.at[pl.ds(base, BPW)])

    return k


_sc_gather = _make_sc_gather()


@functools.partial(jax.jit, static_argnames=())
def _vq_fused(z_res, z_cur, codebook, prec):
    c2 = pl.pallas_call(
        _c2_body,
        out_shape=jax.ShapeDtypeStruct((1, K), jnp.float32),
    )(codebook)

    zero_probs = jnp.zeros((1, K), jnp.float32)
    zero_plogp = jnp.zeros((1, 1), jnp.float32)

    idx_a, probs_a, plogp_a = _tc_half_a(
        z_res[:NH], codebook, c2, prec, zero_probs, zero_plogp)
    idx_b, kld, perp = _tc_half_b(
        z_res[NH:], codebook, c2, prec, probs_a, plogp_a)

    zq_a, zcn_a, zrn_a = _sc_gather(
        codebook, idx_a.reshape(NH), z_cur[:NH], z_res[:NH])
    zq_b, zcn_b, zrn_b = _sc_gather(
        codebook, idx_b.reshape(NH), z_cur[NH:], z_res[NH:])

    z_q = jnp.concatenate([zq_a, zq_b], axis=0)
    z_cur_new = jnp.concatenate([zcn_a, zcn_b], axis=0)
    z_res_new = jnp.concatenate([zrn_a, zrn_b], axis=0)
    return z_cur_new, z_res_new, z_q, kld, perp


def kernel(z_cur, z_res, codebook, log_param_q_scalar_q, flg_train, flg_quant_det):
    del flg_train, flg_quant_det
    prec = (0.5 / jnp.exp(log_param_q_scalar_q)).reshape(1, 1).astype(jnp.float32)
    zr = z_res.reshape(N, D)
    zc_ = z_cur.reshape(N, D)
    z_cur_new, z_res_new, z_q, kld, perp = _vq_fused(zr, zc_, codebook, prec)
    return (z_cur_new.reshape(B, T, D),
            z_res_new.reshape(B, T, D),
            z_q.reshape(B, T, D),
            kld[0, 0],
            perp[0, 0])


# speedup vs baseline: 1.0677x; 1.0677x over previous
import functools

import jax
import jax.numpy as jnp
from jax import lax
from jax.experimental import pallas as pl
from jax.experimental.pallas import tpu as pltpu

B, T, D, K = 4, 1024, 256, 8192
N = B * T
TB = 256
NSTEPS = N // TB


def _vq_body(prec_ref, z_ref, zcur_ref, cb_ref, cbt_ref, cb16_ref,
             zcur_out, zres_out, zq_out, kld_out, perp_out,
             probs_acc, plogp_acc):
    i = pl.program_id(0)

    @pl.when(i == 0)
    def _init():
        probs_acc[...] = jnp.zeros_like(probs_acc)
        plogp_acc[...] = jnp.zeros_like(plogp_acc)

    prec = prec_ref[0, 0]
    z = z_ref[...]
    c = cb_ref[...]
    ct = cbt_ref[...]
    ones_d = jnp.ones((D, 1), jnp.float32)
    ones_k = jnp.ones((K, 1), jnp.float32)

    zc = lax.dot_general(z, c, (((1,), (1,)), ((), ())),
                         preferred_element_type=jnp.float32)
    z2 = lax.dot_general(z * z, ones_d, (((1,), (0,)), ((), ())),
                         preferred_element_type=jnp.float32)
    c2 = jnp.sum(ct * ct, axis=0, keepdims=True)
    dist = z2 - 2.0 * zc + c2
    logits = -prec * dist

    m = jnp.max(logits, axis=1, keepdims=True)
    iota1 = lax.broadcasted_iota(jnp.int32, (1, K), 1)
    idx = jnp.min(jnp.where(logits == m, iota1, K), axis=1)

    t = logits - m
    e = jnp.exp(t)
    et = e * t
    s = lax.dot_general(e, ones_k, (((1,), (0,)), ((), ())),
                        preferred_element_type=jnp.float32)
    set_ = lax.dot_general(et, ones_k, (((1,), (0,)), ((), ())),
                           preferred_element_type=jnp.float32)
    rinv = 1.0 / s
    row_kld = set_ * rinv - jnp.log(s) + jnp.log(float(K))

    plogp_acc[...] = plogp_acc[...] + jnp.sum(row_kld)
    probs_acc[...] += lax.dot_general(rinv, e, (((0,), (0,)), ((), ())),
                                      preferred_element_type=jnp.float32)

    onehot = (iota1 == idx[:, None]).astype(jnp.bfloat16)
    zq = lax.dot_general(onehot, cb16_ref[...], (((1,), (0,)), ((), ())),
                         preferred_element_type=jnp.float32)
    zq_out[...] = zq
    zcur_out[...] = zcur_ref[...] + zq
    zres_out[...] = z - zq

    @pl.when(i == NSTEPS - 1)
    def _fin():
        avg = probs_acc[...] / float(N)
        kld_out[...] = plogp_acc[...] / float(N)
        perp_out[...] = jnp.zeros_like(perp_out) + jnp.exp(
            -jnp.sum(avg * jnp.log(avg + 1e-7)))


@functools.partial(jax.jit, static_argnames=())
def _vq_fused(z_res, z_cur, codebook, cbt, cb16, prec):
    out = pl.pallas_call(
        _vq_body,
        grid=(NSTEPS,),
        in_specs=[
            pl.BlockSpec(memory_space=pltpu.SMEM),
            pl.BlockSpec((TB, D), lambda i: (i, 0)),
            pl.BlockSpec((TB, D), lambda i: (i, 0)),
            pl.BlockSpec((K, D), lambda i: (0, 0)),
            pl.BlockSpec((D, K), lambda i: (0, 0)),
            pl.BlockSpec((K, D), lambda i: (0, 0)),
        ],
        out_specs=[
            pl.BlockSpec((TB, D), lambda i: (i, 0)),
            pl.BlockSpec((TB, D), lambda i: (i, 0)),
            pl.BlockSpec((TB, D), lambda i: (i, 0)),
            pl.BlockSpec((1, 1), lambda i: (0, 0)),
            pl.BlockSpec((1, 1), lambda i: (0, 0)),
        ],
        out_shape=[
            jax.ShapeDtypeStruct((N, D), jnp.float32),
            jax.ShapeDtypeStruct((N, D), jnp.float32),
            jax.ShapeDtypeStruct((N, D), jnp.float32),
            jax.ShapeDtypeStruct((1, 1), jnp.float32),
            jax.ShapeDtypeStruct((1, 1), jnp.float32),
        ],
        scratch_shapes=[
            pltpu.VMEM((1, K), jnp.float32),
            pltpu.VMEM((1, 1), jnp.float32),
        ],
        compiler_params=pltpu.CompilerParams(
            dimension_semantics=("arbitrary",),
        ),
    )(prec, z_res, z_cur, codebook, cbt, cb16)
    return out


def kernel(z_cur, z_res, codebook, log_param_q_scalar_q, flg_train, flg_quant_det):
    del flg_train, flg_quant_det
    prec = (0.5 / jnp.exp(log_param_q_scalar_q)).reshape(1, 1).astype(jnp.float32)
    zr = z_res.reshape(N, D)
    zc_ = z_cur.reshape(N, D)
    cbt = codebook.T
    cb16 = codebook.astype(jnp.bfloat16)
    z_cur_new, z_res_new, z_q, kld, perp = _vq_fused(zr, zc_, codebook, cbt, cb16, prec)
    return (z_cur_new.reshape(B, T, D),
            z_res_new.reshape(B, T, D),
            z_q.reshape(B, T, D),
            kld[0, 0],
            perp[0, 0])


# speedup vs baseline: 1.1897x; 1.1143x over previous
import functools

import jax
import jax.numpy as jnp
from jax import lax
from jax.experimental import pallas as pl
from jax.experimental.pallas import tpu as pltpu

B, T, D, K = 4, 1024, 256, 8192
N = B * T
TB = 256
NSTEPS = N // TB


def _vq_body(prec_ref, z_ref, zcur_ref, cb_ref, cb16_ref,
             zcur_out, zres_out, zq_out, kld_out, perp_out,
             probs_acc, plogp_acc):
    i = pl.program_id(0)

    @pl.when(i == 0)
    def _init():
        probs_acc[...] = jnp.zeros_like(probs_acc)
        plogp_acc[...] = jnp.zeros_like(plogp_acc)

    prec = prec_ref[0, 0]
    z = z_ref[...]
    c = cb_ref[...]
    ones_d = jnp.ones((D, 1), jnp.float32)
    ones_k = jnp.ones((K, 1), jnp.float32)

    zc = lax.dot_general(z, c, (((1,), (1,)), ((), ())),
                         preferred_element_type=jnp.float32)
    z2 = lax.dot_general(z * z, ones_d, (((1,), (0,)), ((), ())),
                         preferred_element_type=jnp.float32)
    c2 = jnp.sum(c * c, axis=1)[None, :]
    dist = z2 - 2.0 * zc + c2
    logits = -prec * dist

    m = jnp.max(logits, axis=1, keepdims=True)
    iota1 = lax.broadcasted_iota(jnp.int32, (1, K), 1)
    idx = jnp.min(jnp.where(logits == m, iota1, K), axis=1)

    t = logits - m
    e = jnp.exp(t)
    et = e * t
    s = lax.dot_general(e, ones_k, (((1,), (0,)), ((), ())),
                        preferred_element_type=jnp.float32)
    set_ = lax.dot_general(et, ones_k, (((1,), (0,)), ((), ())),
                           preferred_element_type=jnp.float32)
    rinv = 1.0 / s
    row_kld = set_ * rinv - jnp.log(s) + jnp.log(float(K))

    plogp_acc[...] = plogp_acc[...] + jnp.sum(row_kld)
    probs_acc[...] += lax.dot_general(rinv, e, (((0,), (0,)), ((), ())),
                                      preferred_element_type=jnp.float32)

    onehot = (iota1 == idx[:, None]).astype(jnp.bfloat16)
    zq = lax.dot_general(onehot, cb16_ref[...], (((1,), (0,)), ((), ())),
                         preferred_element_type=jnp.float32)
    zq_out[...] = zq
    zcur_out[...] = zcur_ref[...] + zq
    zres_out[...] = z - zq

    @pl.when(i == NSTEPS - 1)
    def _fin():
        avg = probs_acc[...] / float(N)
        kld_out[...] = plogp_acc[...] / float(N)
        perp_out[...] = jnp.zeros_like(perp_out) + jnp.exp(
            -jnp.sum(avg * jnp.log(avg + 1e-7)))


@functools.partial(jax.jit, static_argnames=())
def _vq_fused(z_res, z_cur, codebook, cb16, prec):
    out = pl.pallas_call(
        _vq_body,
        grid=(NSTEPS,),
        in_specs=[
            pl.BlockSpec(memory_space=pltpu.SMEM),
            pl.BlockSpec((TB, D), lambda i: (i, 0)),
            pl.BlockSpec((TB, D), lambda i: (i, 0)),
            pl.BlockSpec((K, D), lambda i: (0, 0)),
            pl.BlockSpec((K, D), lambda i: (0, 0)),
        ],
        out_specs=[
            pl.BlockSpec((TB, D), lambda i: (i, 0)),
            pl.BlockSpec((TB, D), lambda i: (i, 0)),
            pl.BlockSpec((TB, D), lambda i: (i, 0)),
            pl.BlockSpec((1, 1), lambda i: (0, 0)),
            pl.BlockSpec((1, 1), lambda i: (0, 0)),
        ],
        out_shape=[
            jax.ShapeDtypeStruct((N, D), jnp.float32),
            jax.ShapeDtypeStruct((N, D), jnp.float32),
            jax.ShapeDtypeStruct((N, D), jnp.float32),
            jax.ShapeDtypeStruct((1, 1), jnp.float32),
            jax.ShapeDtypeStruct((1, 1), jnp.float32),
        ],
        scratch_shapes=[
            pltpu.VMEM((1, K), jnp.float32),
            pltpu.VMEM((1, 1), jnp.float32),
        ],
        compiler_params=pltpu.CompilerParams(
            dimension_semantics=("arbitrary",),
        ),
    )(prec, z_res, z_cur, codebook, cb16)
    return out


def kernel(z_cur, z_res, codebook, log_param_q_scalar_q, flg_train, flg_quant_det):
    del flg_train, flg_quant_det
    prec = (0.5 / jnp.exp(log_param_q_scalar_q)).reshape(1, 1).astype(jnp.float32)
    zr = z_res.reshape(N, D)
    zc_ = z_cur.reshape(N, D)
    cb16 = codebook.astype(jnp.bfloat16)
    z_cur_new, z_res_new, z_q, kld, perp = _vq_fused(zr, zc_, codebook, cb16, prec)
    return (z_cur_new.reshape(B, T, D),
            z_res_new.reshape(B, T, D),
            z_q.reshape(B, T, D),
            kld[0, 0],
            perp[0, 0])
